# baseline (speedup 1.0000x reference)
import functools

import jax
import jax.numpy as jnp
from jax import lax
from jax.experimental import pallas as pl
from jax.experimental.pallas import tpu as pltpu
from jax.experimental.pallas import tpu_sc as plsc

NUM_USERS = 1000000
NUM_ITEMS = 1000000
EMB_DIM = 32
BATCH = 16384

NC = 2
NS = 16
NW = NC * NS
BPW = BATCH // NW
CHUNK = 128
NCHUNK = BPW // CHUNK
LANES = 16
GROUPS = BPW // LANES

_mesh = plsc.VectorSubcoreMesh(
    core_axis_name="c", subcore_axis_name="s", num_cores=NC, num_subcores=NS
)


@functools.partial(
    pl.kernel,
    out_type=jax.ShapeDtypeStruct((BATCH,), jnp.float32),
    mesh=_mesh,
    scratch_types=[
        pltpu.VMEM((BPW,), jnp.int32),
        pltpu.VMEM((BPW,), jnp.int32),
        pltpu.VMEM((BPW, EMB_DIM), jnp.float32),
        pltpu.VMEM((BPW, EMB_DIM), jnp.float32),
        pltpu.VMEM((BPW,), jnp.float32),
        pltpu.SemaphoreType.DMA,
    ],
    compiler_params=pltpu.CompilerParams(
        needs_layout_passes=False, use_tc_tiling_on_sc=False),
)
def _lightgcn_sc(uid_hbm, iid_hbm, utab_hbm, itab_hbm, out_hbm,
                 uidx_v, iidx_v, urows_v, irows_v, out_v, sem):
    wid = lax.axis_index("s") * NC + lax.axis_index("c")
    base = wid * BPW

    pltpu.sync_copy(uid_hbm.at[pl.ds(base, BPW)], uidx_v)
    pltpu.sync_copy(iid_hbm.at[pl.ds(base, BPW)], iidx_v)

    copies = []
    for j in range(NCHUNK):
        sl = pl.ds(j * CHUNK, CHUNK)
        copies.append(
            pltpu.async_copy(utab_hbm.at[uidx_v.at[sl]], urows_v.at[sl], sem))
        copies.append(
            pltpu.async_copy(itab_hbm.at[iidx_v.at[sl]], irows_v.at[sl], sem))
    for cp in copies:
        cp.wait()

    lane_iota = lax.iota(jnp.int32, LANES)

    def group_body(g, carry):
        row0 = g * LANES
        row_idx = row0 + lane_iota
        acc = jnp.zeros((LANES,), jnp.float32)
        for d in range(EMB_DIM):
            col_idx = jnp.full((LANES,), d, jnp.int32)
            u = plsc.load_gather(urows_v, [row_idx, col_idx])
            v = plsc.load_gather(irows_v, [row_idx, col_idx])
            acc = acc + u * v
        out_v[pl.ds(row0, LANES)] = acc
        return carry

    lax.fori_loop(0, GROUPS, group_body, None)

    pltpu.sync_copy(out_v, out_hbm.at[pl.ds(base, BPW)])


def kernel(user_ids, item_ids, user_embeddings, item_embeddings):
    return _lightgcn_sc(
        user_ids.astype(jnp.int32),
        item_ids.astype(jnp.int32),
        user_embeddings,
        item_embeddings,
    )
